# baseline (speedup 1.0000x reference)
import functools

import jax
import jax.numpy as jnp
from jax import lax
from jax.experimental import pallas as pl
from jax.experimental.pallas import tpu as pltpu
from jax.experimental.pallas import tpu_sc as plsc

LOC_DIM = 64
ACT_DIM = 32
FEAT_DIM = 16
FEAT_EMB_DIM = 32
OUT_DIM = 128

NUM_WORKERS = 32
CHUNK = 128
DEPTH = 4


def _feat_proj_tc(features2d, Wt, b2d, block_n=4096):
    n = features2d.shape[0]

    def body(x_ref, w_ref, b_ref, o_ref):
        o_ref[...] = (
            jnp.dot(x_ref[...], w_ref[...], preferred_element_type=jnp.float32)
            + b_ref[...]
        )

    return pl.pallas_call(
        body,
        grid=(n // block_n,),
        in_specs=[
            pl.BlockSpec((block_n, FEAT_DIM), lambda i: (i, 0)),
            pl.BlockSpec((FEAT_DIM, FEAT_EMB_DIM), lambda i: (0, 0)),
            pl.BlockSpec((1, FEAT_EMB_DIM), lambda i: (0, 0)),
        ],
        out_specs=pl.BlockSpec((block_n, FEAT_EMB_DIM), lambda i: (i, 0)),
        out_shape=jax.ShapeDtypeStruct((n, FEAT_EMB_DIM), jnp.float32),
    )(features2d, Wt, b2d)


def _assemble_sc(tok2, feat_emb, loc_table, act_table):
    n = feat_emb.shape[0]
    per_w = n // NUM_WORKERS
    steps = per_w // CHUNK
    ng = steps // DEPTH
    mesh = plsc.VectorSubcoreMesh(core_axis_name="c", subcore_axis_name="s")

    buf_set = [
        pltpu.VMEM((1, 2, 128), jnp.int32),
        pltpu.VMEM((CHUNK, LOC_DIM), jnp.float32),
        pltpu.VMEM((CHUNK, ACT_DIM), jnp.float32),
        pltpu.VMEM((CHUNK, FEAT_EMB_DIM), jnp.float32),
        pltpu.SemaphoreType.DMA,
        pltpu.SemaphoreType.DMA,
        pltpu.SemaphoreType.DMA,
    ]

    @functools.partial(
        pl.kernel,
        out_type=jax.ShapeDtypeStruct((n, OUT_DIM), jnp.float32),
        mesh=mesh,
        scratch_types=buf_set * DEPTH,
        compiler_params=pltpu.CompilerParams(use_tc_tiling_on_sc=False),
    )
    def k(tok_hbm, feat_hbm, ltab_hbm, atab_hbm, out_hbm, *scr):
        sets = [scr[7 * i:7 * i + 7] for i in range(DEPTH)]
        wid = lax.axis_index("s") * 2 + lax.axis_index("c")
        row0 = wid * steps
        tok0 = wid * per_w

        def idx_cp(S, c):
            return [pltpu.make_async_copy(
                tok_hbm.at[pl.ds(row0 + c, 1)], S[0], S[4])]

        def gathers(S, c):
            tb = tok0 + c * CHUNK
            return [
                pltpu.make_async_copy(ltab_hbm.at[S[0].at[0, 0]], S[1], S[5]),
                pltpu.make_async_copy(atab_hbm.at[S[0].at[0, 1]], S[2], S[5]),
                pltpu.make_async_copy(feat_hbm.at[pl.ds(tb, CHUNK)], S[3], S[5]),
            ]

        def writes(S, c):
            tb = tok0 + c * CHUNK
            return [
                pltpu.make_async_copy(
                    S[1], out_hbm.at[pl.ds(tb, CHUNK), pl.ds(0, LOC_DIM)],
                    S[6]),
                pltpu.make_async_copy(
                    S[2], out_hbm.at[pl.ds(tb, CHUNK), pl.ds(LOC_DIM, ACT_DIM)],
                    S[6]),
                pltpu.make_async_copy(
                    S[3], out_hbm.at[pl.ds(tb, CHUNK),
                                     pl.ds(LOC_DIM + ACT_DIM, FEAT_EMB_DIM)],
                    S[6]),
            ]

        def start(cps):
            for cp in cps:
                cp.start()

        def wait(cps):
            for cp in cps:
                cp.wait()

        for d in range(DEPTH):
            start(idx_cp(sets[d], d))

        def body(g, carry):
            for d in range(DEPTH):
                c = DEPTH * g + d
                S = sets[d]
                Sg = sets[(d - 2) % DEPTH]

                def harvest(c=c, Sg=Sg):
                    wait(gathers(Sg, c - 2))
                    start(writes(Sg, c - 2))

                def refill(c=c, Sg=Sg):
                    start(idx_cp(Sg, c + 2))

                if d >= 2:
                    harvest()

                    @pl.when(g < ng - 1)
                    def _():
                        refill()
                else:
                    @pl.when(g > 0)
                    def _():
                        harvest()
                        refill()

                @pl.when(g > 0)
                def _():
                    wait(writes(S, c - DEPTH))

                wait(idx_cp(S, c))
                start(gathers(S, c))
            return carry

        lax.fori_loop(0, ng, body, 0)

        for c in (steps - 2, steps - 1):
            Sg = sets[c % DEPTH]
            wait(gathers(Sg, c))
            start(writes(Sg, c))
        for c in range(steps - DEPTH, steps):
            wait(writes(sets[c % DEPTH], c))

    return k(tok2, feat_emb, loc_table, act_table)


def kernel(loc_tokens, act_tokens, features, loc_table, act_table, W, b):
    bsz, seq = loc_tokens.shape
    n = bsz * seq
    feat_emb = _feat_proj_tc(
        features.reshape(n, FEAT_DIM), W.T, b.reshape(1, FEAT_EMB_DIM))
    lt2 = loc_tokens.reshape(n // 128, 128).astype(jnp.int32)
    at2 = act_tokens.reshape(n // 128, 128).astype(jnp.int32)
    tok2 = jnp.stack([lt2, at2], axis=1)
    out = _assemble_sc(tok2, feat_emb, loc_table, act_table)
    return out.reshape(bsz, seq, OUT_DIM)


# speedup vs baseline: 1.0034x; 1.0034x over previous
import functools

import jax
import jax.numpy as jnp
from jax import lax
from jax.experimental import pallas as pl
from jax.experimental.pallas import tpu as pltpu
from jax.experimental.pallas import tpu_sc as plsc

LOC_DIM = 64
ACT_DIM = 32
FEAT_DIM = 16
FEAT_EMB_DIM = 32
OUT_DIM = 128

NUM_WORKERS = 32
TK = 2
CHUNK = TK * 128


def _feat_proj_tc(features2d, Wt, b2d, block_n=4096):
    n = features2d.shape[0]

    def body(x_ref, w_ref, b_ref, o_ref):
        o_ref[...] = (
            jnp.dot(x_ref[...], w_ref[...], preferred_element_type=jnp.float32)
            + b_ref[...]
        )

    return pl.pallas_call(
        body,
        grid=(n // block_n,),
        in_specs=[
            pl.BlockSpec((block_n, FEAT_DIM), lambda i: (i, 0)),
            pl.BlockSpec((FEAT_DIM, FEAT_EMB_DIM), lambda i: (0, 0)),
            pl.BlockSpec((1, FEAT_EMB_DIM), lambda i: (0, 0)),
        ],
        out_specs=pl.BlockSpec((block_n, FEAT_EMB_DIM), lambda i: (i, 0)),
        out_shape=jax.ShapeDtypeStruct((n, FEAT_EMB_DIM), jnp.float32),
    )(features2d, Wt, b2d)


def _assemble_sc(loc_tok2d, act_tok2d, feat_emb, loc_table, act_table):
    n = feat_emb.shape[0]
    per_w = n // NUM_WORKERS
    rows_w = per_w // 128
    steps = per_w // CHUNK
    ng = steps // 2
    mesh = plsc.VectorSubcoreMesh(core_axis_name="c", subcore_axis_name="s")

    buf_set = [
        pltpu.VMEM((CHUNK,), jnp.int32),
        pltpu.VMEM((CHUNK,), jnp.int32),
        pltpu.VMEM((CHUNK, LOC_DIM), jnp.float32),
        pltpu.VMEM((CHUNK, ACT_DIM), jnp.float32),
        pltpu.VMEM((CHUNK, FEAT_EMB_DIM), jnp.float32),
        pltpu.SemaphoreType.DMA,
        pltpu.SemaphoreType.DMA,
        pltpu.SemaphoreType.DMA,
    ]
    n_act = 1004

    @functools.partial(
        pl.kernel,
        out_type=jax.ShapeDtypeStruct((n, OUT_DIM), jnp.float32),
        mesh=mesh,
        scratch_types=buf_set + buf_set,
        compiler_params=pltpu.CompilerParams(use_tc_tiling_on_sc=False),
    )
    def k(loc_hbm, act_hbm, feat_hbm, ltab_hbm, atab_hbm, out_hbm, *scr):
        A, B = scr[:8], scr[8:16]
        sid = lax.axis_index("s")
        wid = sid * 2 + lax.axis_index("c")
        row0 = wid * rows_w
        tok0 = wid * per_w

        def idx_cp(S, c):
            tb = tok0 + c * CHUNK
            return [
                pltpu.make_async_copy(loc_hbm.at[pl.ds(tb, CHUNK)], S[0], S[5]),
                pltpu.make_async_copy(act_hbm.at[pl.ds(tb, CHUNK)], S[1], S[5]),
            ]

        def gathers(S, c):
            tb = tok0 + c * CHUNK
            return [
                pltpu.make_async_copy(ltab_hbm.at[S[0]], S[2], S[6]),
                pltpu.make_async_copy(atab_hbm.at[S[1]], S[3], S[6]),
                pltpu.make_async_copy(feat_hbm.at[pl.ds(tb, CHUNK)], S[4], S[6]),
            ]

        def writes(S, c):
            tb = tok0 + c * CHUNK
            return [
                pltpu.make_async_copy(
                    S[2], out_hbm.at[pl.ds(tb, CHUNK), pl.ds(0, LOC_DIM)], S[7]),
                pltpu.make_async_copy(
                    S[3], out_hbm.at[pl.ds(tb, CHUNK), pl.ds(LOC_DIM, ACT_DIM)], S[7]),
                pltpu.make_async_copy(
                    S[4], out_hbm.at[pl.ds(tb, CHUNK),
                                     pl.ds(LOC_DIM + ACT_DIM, FEAT_EMB_DIM)], S[7]),
            ]

        def start(cps):
            for cp in cps:
                cp.start()

        def wait(cps):
            for cp in cps:
                cp.wait()

        start(idx_cp(A, 0))
        wait(idx_cp(A, 0))
        start(gathers(A, 0))
        start(idx_cp(B, 1))

        def body(g, carry):
            a = 2 * g
            b = a + 1

            @pl.when(g > 0)
            def _():
                wait(writes(B, b))

            wait(idx_cp(B, b))
            start(gathers(B, b))
            wait(gathers(A, a))

            @pl.when(g < ng - 1)
            def _():
                start(idx_cp(A, a + 2))

            start(writes(A, a))
            wait(gathers(B, b))

            @pl.when(g < ng - 1)
            def _():
                start(idx_cp(B, b + 2))

            start(writes(B, b))

            @pl.when(g < ng - 1)
            def _():
                wait(writes(A, a))
                wait(idx_cp(A, a + 2))
                start(gathers(A, a + 2))

            @pl.when(g == ng - 1)
            def _():
                wait(writes(A, a))

            return carry

        lax.fori_loop(0, ng, body, 0)
        wait(writes(B, 1))

    return k(loc_tok2d, act_tok2d, feat_emb, loc_table, act_table)


def kernel(loc_tokens, act_tokens, features, loc_table, act_table, W, b):
    bsz, seq = loc_tokens.shape
    n = bsz * seq
    feat_emb = _feat_proj_tc(
        features.reshape(n, FEAT_DIM), W.T, b.reshape(1, FEAT_EMB_DIM))
    lt2 = loc_tokens.reshape(n).astype(jnp.int32)
    at2 = act_tokens.reshape(n).astype(jnp.int32)
    out = _assemble_sc(lt2, at2, feat_emb, loc_table, act_table)
    return out.reshape(bsz, seq, OUT_DIM)


# speedup vs baseline: 1.0106x; 1.0072x over previous
import functools

import jax
import jax.numpy as jnp
from jax import lax
from jax.experimental import pallas as pl
from jax.experimental.pallas import tpu as pltpu
from jax.experimental.pallas import tpu_sc as plsc

LOC_DIM = 64
ACT_DIM = 32
FEAT_DIM = 16
FEAT_EMB_DIM = 32
OUT_DIM = 128

NUM_WORKERS = 32
CHUNK = 128
NDATA = 4
NIDX = 8


def _feat_proj_tc(features2d, Wt, b2d, block_n=4096):
    n = features2d.shape[0]

    def body(x_ref, w_ref, b_ref, o_ref):
        o_ref[...] = (
            jnp.dot(x_ref[...], w_ref[...], preferred_element_type=jnp.float32)
            + b_ref[...]
        )

    return pl.pallas_call(
        body,
        grid=(n // block_n,),
        in_specs=[
            pl.BlockSpec((block_n, FEAT_DIM), lambda i: (i, 0)),
            pl.BlockSpec((FEAT_DIM, FEAT_EMB_DIM), lambda i: (0, 0)),
            pl.BlockSpec((1, FEAT_EMB_DIM), lambda i: (0, 0)),
        ],
        out_specs=pl.BlockSpec((block_n, FEAT_EMB_DIM), lambda i: (i, 0)),
        out_shape=jax.ShapeDtypeStruct((n, FEAT_EMB_DIM), jnp.float32),
    )(features2d, Wt, b2d)


def _assemble_sc(loc_tok2d, act_tok2d, feat_emb, loc_table, act_table):
    n = feat_emb.shape[0]
    per_w = n // NUM_WORKERS
    steps = per_w // CHUNK
    ng = steps // NIDX
    mesh = plsc.VectorSubcoreMesh(core_axis_name="c", subcore_axis_name="s")

    data_set = [
        pltpu.VMEM((CHUNK, LOC_DIM), jnp.float32),
        pltpu.VMEM((CHUNK, ACT_DIM), jnp.float32),
        pltpu.VMEM((CHUNK, FEAT_EMB_DIM), jnp.float32),
        pltpu.SemaphoreType.DMA,
        pltpu.SemaphoreType.DMA,
    ]
    idx_set = [
        pltpu.VMEM((CHUNK,), jnp.int32),
        pltpu.VMEM((CHUNK,), jnp.int32),
        pltpu.SemaphoreType.DMA,
    ]

    @functools.partial(
        pl.kernel,
        out_type=jax.ShapeDtypeStruct((n, OUT_DIM), jnp.float32),
        mesh=mesh,
        scratch_types=data_set * NDATA + idx_set * NIDX,
        compiler_params=pltpu.CompilerParams(use_tc_tiling_on_sc=False),
    )
    def k(loc_hbm, act_hbm, feat_hbm, ltab_hbm, atab_hbm, out_hbm, *scr):
        D = [scr[5 * i:5 * i + 5] for i in range(NDATA)]
        I = [scr[5 * NDATA + 3 * i:5 * NDATA + 3 * i + 3] for i in range(NIDX)]
        sid = lax.axis_index("s")
        wid = sid * 2 + lax.axis_index("c")
        tok0 = wid * per_w

        def idx_cp(Ik, c):
            tb = tok0 + c * CHUNK
            return [
                pltpu.make_async_copy(loc_hbm.at[pl.ds(tb, CHUNK)], Ik[0], Ik[2]),
                pltpu.make_async_copy(act_hbm.at[pl.ds(tb, CHUNK)], Ik[1], Ik[2]),
            ]

        def gathers(S, Ik, c):
            tb = tok0 + c * CHUNK
            return [
                pltpu.make_async_copy(ltab_hbm.at[Ik[0]], S[0], S[3]),
                pltpu.make_async_copy(atab_hbm.at[Ik[1]], S[1], S[3]),
                pltpu.make_async_copy(feat_hbm.at[pl.ds(tb, CHUNK)], S[2], S[3]),
            ]

        def writes(S, c):
            tb = tok0 + c * CHUNK
            return [
                pltpu.make_async_copy(
                    S[0], out_hbm.at[pl.ds(tb, CHUNK), pl.ds(0, LOC_DIM)], S[4]),
                pltpu.make_async_copy(
                    S[1], out_hbm.at[pl.ds(tb, CHUNK), pl.ds(LOC_DIM, ACT_DIM)], S[4]),
                pltpu.make_async_copy(
                    S[2], out_hbm.at[pl.ds(tb, CHUNK),
                                     pl.ds(LOC_DIM + ACT_DIM, FEAT_EMB_DIM)], S[4]),
            ]

        def start(cps):
            for cp in cps:
                cp.start()

        def wait(cps):
            for cp in cps:
                cp.wait()

        for d in range(NIDX):
            start(idx_cp(I[d], d))

        def body(h, carry):
            for d in range(NIDX):
                c = NIDX * h + d
                S = D[d % NDATA]
                Sp = D[(d - 2) % NDATA]
                Ic = I[d]
                Ip = I[(d - 2) % NIDX]

                def drain(c=c, S=S):
                    wait(writes(S, c - NDATA))

                if d >= NDATA:
                    drain()
                else:
                    @pl.when(h > 0)
                    def _():
                        drain()

                wait(idx_cp(Ic, c))
                start(gathers(S, Ic, c))

                def harvest(c=c, Sp=Sp):
                    wait(gathers(Sp, I[(d - 2) % NIDX], c - 2))
                    start(writes(Sp, c - 2))

                def refill(c=c, Ip=Ip):
                    start(idx_cp(Ip, c + 6))

                if d >= 2:
                    harvest()

                    @pl.when(h < ng - 1)
                    def _():
                        refill()
                else:
                    @pl.when(h > 0)
                    def _():
                        harvest()
                        refill()
            return carry

        lax.fori_loop(0, ng, body, 0)

        for c in (steps - 2, steps - 1):
            wait(gathers(D[c % NDATA], I[c % NIDX], c))
            start(writes(D[c % NDATA], c))
        for c in range(steps - NDATA, steps):
            wait(writes(D[c % NDATA], c))

    return k(loc_tok2d, act_tok2d, feat_emb, loc_table, act_table)


def kernel(loc_tokens, act_tokens, features, loc_table, act_table, W, b):
    bsz, seq = loc_tokens.shape
    n = bsz * seq
    feat_emb = _feat_proj_tc(
        features.reshape(n, FEAT_DIM), W.T, b.reshape(1, FEAT_EMB_DIM))
    lt2 = loc_tokens.reshape(n).astype(jnp.int32)
    at2 = act_tokens.reshape(n).astype(jnp.int32)
    out = _assemble_sc(lt2, at2, feat_emb, loc_table, act_table)
    return out.reshape(bsz, seq, OUT_DIM)
